# baseline (speedup 1.0000x reference)
import functools

import jax
import jax.numpy as jnp
from jax import lax
from jax.experimental import pallas as pl
from jax.experimental.pallas import tpu as pltpu
from jax.experimental.pallas import tpu_sc as plsc


def _make_k2(V4, B0, B1, D, NW):
    mesh = plsc.VectorSubcoreMesh(core_axis_name="c", subcore_axis_name="s")
    NC = 2
    npw = B0 // NW
    NBB = npw // 128
    NU = B1 * NBB
    NI = NU // 2

    @functools.partial(
        pl.kernel,
        out_type=jax.ShapeDtypeStruct((B1, D, B0), jnp.float32),
        mesh=mesh,
        scratch_types=[
            pltpu.VMEM((B1, npw), jnp.int32),
            pltpu.VMEM((128,), jnp.int32),
            pltpu.VMEM((128,), jnp.int32),
            pltpu.VMEM((128, 128), jnp.float32),
            pltpu.VMEM((128, 128), jnp.float32),
            pltpu.VMEM((D, 128), jnp.float32),
            pltpu.VMEM((D, 128), jnp.float32),
            pltpu.SemaphoreType.DMA,
            pltpu.SemaphoreType.DMA,
            pltpu.SemaphoreType.DMA,
            pltpu.SemaphoreType.DMA,
        ],
        compiler_params=pltpu.CompilerParams(
            use_tc_tiling_on_sc=True, needs_layout_passes=False
        ),
    )
    def k2(t4_hbm, idxT_hbm, outT_hbm, islab, *bufs):
        off = bufs[0:2]
        gbuf = bufs[2:4]
        tbuf = bufs[4:6]
        semg = bufs[6:8]
        semo = bufs[8:10]
        wid = lax.axis_index("s") * NC + lax.axis_index("c")
        col0 = wid * npw
        pltpu.sync_copy(idxT_hbm.at[:, pl.ds(col0, npw)], islab)
        lanes = jnp.arange(16, dtype=jnp.int32)

        def prep(u, p):
            s = u // NBB
            boff = (u % NBB) * 128
            for k in range(8):
                v = islab[s, pl.ds(boff + 16 * k, 16)]
                off[p][pl.ds(16 * k, 16)] = v >> 2

        def gather_start(p):
            pltpu.async_copy(t4_hbm.at[off[p]], gbuf[p], semg[p])

        def gather_wait(p):
            pltpu.make_async_copy(t4_hbm.at[off[p]], gbuf[p], semg[p]).wait()

        def transform(u, p):
            s = u // NBB
            boff = (u % NBB) * 128
            for l in range(8):
                idx16 = islab[s, pl.ds(boff + 16 * l, 16)]
                cbase = (idx16 & 3) * D
                row = lanes + (16 * l)
                for e in range(D):
                    vreg = plsc.load_gather(gbuf[p], [row, cbase + e])
                    tbuf[p][e, pl.ds(16 * l, 16)] = vreg

        def store_start(u, p):
            s = u // NBB
            boff = (u % NBB) * 128
            pltpu.async_copy(
                tbuf[p], outT_hbm.at[s, :, pl.ds(col0 + boff, 128)], semo[p]
            )

        def store_wait(p):
            pltpu.make_async_copy(
                tbuf[p], outT_hbm.at[0, :, pl.ds(0, 128)], semo[p]
            ).wait()

        prep(0, 0)
        gather_start(0)

        def body(i, carry):
            u0 = 2 * i
            u1 = 2 * i + 1
            prep(u1, 1)
            gather_start(1)
            gather_wait(0)

            @pl.when(i > 0)
            def _():
                store_wait(0)

            transform(u0, 0)
            store_start(u0, 0)

            @pl.when(i < NI - 1)
            def _():
                prep(u0 + 2, 0)
                gather_start(0)

            gather_wait(1)

            @pl.when(i > 0)
            def _():
                store_wait(1)

            transform(u1, 1)
            store_start(u1, 1)
            return carry

        lax.fori_loop(0, NI, body, 0, unroll=False)
        store_wait(0)
        store_wait(1)

    return k2


def kernel(input_nodes, table):
    B0, B1 = input_nodes.shape
    V, D = table.shape
    t4 = table.reshape(V // 4, 4 * D)
    idxT = input_nodes.T.astype(jnp.int32)
    outT = _make_k2(V // 4, B0, B1, D, NW=32)(t4, idxT)
    return jnp.transpose(outT, (2, 0, 1))


# speedup vs baseline: 1.2922x; 1.2922x over previous
import functools

import jax
import jax.numpy as jnp
from jax import lax
from jax.experimental import pallas as pl
from jax.experimental.pallas import tpu as pltpu
from jax.experimental.pallas import tpu_sc as plsc


def _make_k2(V4, B0, B1, D, NW):
    mesh = plsc.VectorSubcoreMesh(core_axis_name="c", subcore_axis_name="s")
    NC = 2
    npw = B0 // NW
    NBB = npw // 128
    NU = B1 * NBB
    NI = NU // 2

    @functools.partial(
        pl.kernel,
        out_type=jax.ShapeDtypeStruct((B1, D, B0), jnp.float32),
        mesh=mesh,
        scratch_types=[
            pltpu.VMEM((B1, npw), jnp.int32),
            pltpu.VMEM((128,), jnp.int32),
            pltpu.VMEM((128,), jnp.int32),
            pltpu.VMEM((128, 128), jnp.float32),
            pltpu.VMEM((128, 128), jnp.float32),
            pltpu.VMEM((D, 128), jnp.float32),
            pltpu.VMEM((D, 128), jnp.float32),
            pltpu.SemaphoreType.DMA,
            pltpu.SemaphoreType.DMA,
            pltpu.SemaphoreType.DMA,
            pltpu.SemaphoreType.DMA,
        ],
        compiler_params=pltpu.CompilerParams(
            use_tc_tiling_on_sc=True, needs_layout_passes=False
        ),
    )
    def k2(t4_hbm, idxT_hbm, outT_hbm, islab, *bufs):
        off = bufs[0:2]
        gbuf = bufs[2:4]
        tbuf = bufs[4:6]
        semg = bufs[6:8]
        semo = bufs[8:10]
        wid = lax.axis_index("s") * NC + lax.axis_index("c")
        col0 = wid * npw
        pltpu.sync_copy(idxT_hbm.at[:, pl.ds(col0, npw)], islab)
        lanes = jnp.arange(16, dtype=jnp.int32)

        def prep(u, p):
            s = u // NBB
            boff = (u % NBB) * 128
            for k in range(8):
                v = islab[s, pl.ds(boff + 16 * k, 16)]
                off[p][pl.ds(16 * k, 16)] = v >> 2

        def gather_start(p):
            pltpu.async_copy(t4_hbm.at[off[p]], gbuf[p], semg[p])

        def gather_wait(p):
            pltpu.make_async_copy(t4_hbm.at[off[p]], gbuf[p], semg[p]).wait()

        def transform(u, p):
            s = u // NBB
            boff = (u % NBB) * 128
            for l in range(8):
                idx16 = islab[s, pl.ds(boff + 16 * l, 16)]
                cbase = (idx16 & 3) * D
                row = lanes + (16 * l)
                vs = [
                    plsc.load_gather(gbuf[p], [row, cbase + e]) for e in range(D)
                ]
                for e in range(D):
                    tbuf[p][e, pl.ds(16 * l, 16)] = vs[e]

        def store_start(u, p):
            s = u // NBB
            boff = (u % NBB) * 128
            pltpu.async_copy(
                tbuf[p], outT_hbm.at[s, :, pl.ds(col0 + boff, 128)], semo[p]
            )

        def store_wait(p):
            pltpu.make_async_copy(
                tbuf[p], outT_hbm.at[0, :, pl.ds(0, 128)], semo[p]
            ).wait()

        prep(0, 0)
        gather_start(0)

        def body(i, carry):
            u0 = 2 * i
            u1 = 2 * i + 1
            prep(u1, 1)
            gather_start(1)
            gather_wait(0)

            @pl.when(i > 0)
            def _():
                store_wait(0)

            transform(u0, 0)
            store_start(u0, 0)

            @pl.when(i < NI - 1)
            def _():
                prep(u0 + 2, 0)
                gather_start(0)

            gather_wait(1)

            @pl.when(i > 0)
            def _():
                store_wait(1)

            transform(u1, 1)
            store_start(u1, 1)
            return carry

        lax.fori_loop(0, NI, body, 0, unroll=False)
        store_wait(0)
        store_wait(1)

    return k2


def kernel(input_nodes, table):
    B0, B1 = input_nodes.shape
    V, D = table.shape
    t4 = table.reshape(V // 4, 4 * D)
    idxT = input_nodes.T.astype(jnp.int32)
    outT = _make_k2(V // 4, B0, B1, D, NW=32)(t4, idxT)
    return jnp.transpose(outT, (2, 0, 1))
